# baseline (speedup 1.0000x reference)
import functools

import jax
import jax.numpy as jnp
from jax.experimental import pallas as pl
from jax.experimental.pallas import tpu as pltpu

_CHUNK = 2048
_NCHUNK = 8192 // _CHUNK
_NBUF = 2


def _linear_kernel(x_hbm, w_ref, b_ref, out_hbm, xbuf, obuf, insem, outsem):
    def in_dma(c, s):
        return pltpu.make_async_copy(
            x_hbm.at[pl.ds(c * _CHUNK, _CHUNK), :], xbuf.at[s], insem.at[s]
        )

    def out_dma(c, s):
        return pltpu.make_async_copy(
            obuf.at[s], out_hbm.at[pl.ds(c * _CHUNK, _CHUNK), :], outsem.at[s]
        )

    for s in range(min(_NBUF, _NCHUNK)):
        in_dma(s, s).start()
    for c in range(_NCHUNK):
        s = c % _NBUF
        in_dma(c, s).wait()
        if c >= _NBUF:
            out_dma(c - _NBUF, s).wait()
        acc = jax.lax.dot_general(
            xbuf[s], w_ref[...], (((1,), (1,)), ((), ())),
            preferred_element_type=jnp.float32,
        )
        obuf[s] = acc + b_ref[...]
        out_dma(c, s).start()
        nxt = c + _NBUF
        if nxt < _NCHUNK:
            in_dma(nxt, s).start()
    for c in range(max(0, _NCHUNK - _NBUF), _NCHUNK):
        out_dma(c, c % _NBUF).wait()


@functools.partial(jax.jit, static_argnames=())
def kernel(x, indices, W, b):
    del indices
    m, k = x.shape
    n = W.shape[0]
    b2 = b.reshape(1, n)
    return pl.pallas_call(
        _linear_kernel,
        in_specs=[
            pl.BlockSpec(memory_space=pltpu.MemorySpace.HBM),
            pl.BlockSpec(memory_space=pltpu.MemorySpace.VMEM),
            pl.BlockSpec(memory_space=pltpu.MemorySpace.VMEM),
        ],
        out_specs=pl.BlockSpec(memory_space=pltpu.MemorySpace.HBM),
        out_shape=jax.ShapeDtypeStruct((m, n), jnp.float32),
        scratch_shapes=[
            pltpu.VMEM((_NBUF, _CHUNK, k), jnp.float32),
            pltpu.VMEM((_NBUF, _CHUNK, n), jnp.float32),
            pltpu.SemaphoreType.DMA((_NBUF,)),
            pltpu.SemaphoreType.DMA((_NBUF,)),
        ],
        compiler_params=pltpu.CompilerParams(
            vmem_limit_bytes=100 * 1024 * 1024,
        ),
    )(x, W, b2)


# speedup vs baseline: 1.1091x; 1.1091x over previous
import functools

import jax
import jax.numpy as jnp
from jax.experimental import pallas as pl
from jax.experimental.pallas import tpu as pltpu

_CHUNK = 1024
_NCHUNK = 8192 // _CHUNK
_NBUF = 4


def _linear_kernel(x_hbm, w_ref, b_ref, out_hbm, xbuf, obuf, insem, outsem):
    def in_dma(c, s):
        return pltpu.make_async_copy(
            x_hbm.at[pl.ds(c * _CHUNK, _CHUNK), :], xbuf.at[s], insem.at[s]
        )

    def out_dma(c, s):
        return pltpu.make_async_copy(
            obuf.at[s], out_hbm.at[pl.ds(c * _CHUNK, _CHUNK), :], outsem.at[s]
        )

    for s in range(min(_NBUF, _NCHUNK)):
        in_dma(s, s).start()
    for c in range(_NCHUNK):
        s = c % _NBUF
        in_dma(c, s).wait()
        if c >= _NBUF:
            out_dma(c - _NBUF, s).wait()
        acc = jax.lax.dot_general(
            xbuf[s], w_ref[...], (((1,), (1,)), ((), ())),
            preferred_element_type=jnp.float32,
        )
        obuf[s] = acc + b_ref[...]
        out_dma(c, s).start()
        nxt = c + _NBUF
        if nxt < _NCHUNK:
            in_dma(nxt, s).start()
    for c in range(max(0, _NCHUNK - _NBUF), _NCHUNK):
        out_dma(c, c % _NBUF).wait()


@functools.partial(jax.jit, static_argnames=())
def kernel(x, indices, W, b):
    del indices
    m, k = x.shape
    n = W.shape[0]
    b2 = b.reshape(1, n)
    return pl.pallas_call(
        _linear_kernel,
        in_specs=[
            pl.BlockSpec(memory_space=pltpu.MemorySpace.HBM),
            pl.BlockSpec(memory_space=pltpu.MemorySpace.VMEM),
            pl.BlockSpec(memory_space=pltpu.MemorySpace.VMEM),
        ],
        out_specs=pl.BlockSpec(memory_space=pltpu.MemorySpace.HBM),
        out_shape=jax.ShapeDtypeStruct((m, n), jnp.float32),
        scratch_shapes=[
            pltpu.VMEM((_NBUF, _CHUNK, k), jnp.float32),
            pltpu.VMEM((_NBUF, _CHUNK, n), jnp.float32),
            pltpu.SemaphoreType.DMA((_NBUF,)),
            pltpu.SemaphoreType.DMA((_NBUF,)),
        ],
        compiler_params=pltpu.CompilerParams(
            vmem_limit_bytes=100 * 1024 * 1024,
        ),
    )(x, W, b2)
